# baseline (speedup 1.0000x reference)
import jax
import jax.numpy as jnp
import numpy as np
from jax.experimental import pallas as pl

_FEAT_STRIDE = 16
_NMS_THRESH = 0.7
_PRE_NMS = 2000
_POST_NMS = 300
_MIN_SIZE = 16.0


def _conv2d(x, w, b, pad):
    out = jax.lax.conv_general_dilated(
        x, w, (1, 1), [(pad, pad), (pad, pad)],
        dimension_numbers=("NCHW", "OIHW", "NCHW"))
    return out + b[None, :, None, None]


def _loc2bbox(anchor, loc):
    ah = anchor[:, 2] - anchor[:, 0]
    aw = anchor[:, 3] - anchor[:, 1]
    acy = anchor[:, 0] + 0.5 * ah
    acx = anchor[:, 1] + 0.5 * aw
    dy, dx, dh, dw = loc[:, 0], loc[:, 1], loc[:, 2], loc[:, 3]
    cy = dy * ah + acy
    cx = dx * aw + acx
    hh = jnp.exp(dh) * ah
    ww = jnp.exp(dw) * aw
    return jnp.stack([cy - 0.5 * hh, cx - 0.5 * ww, cy + 0.5 * hh, cx + 0.5 * ww], axis=1)


def _iou_matrix(boxes):
    area = (boxes[:, 2] - boxes[:, 0]) * (boxes[:, 3] - boxes[:, 1])
    tl = jnp.maximum(boxes[:, None, :2], boxes[None, :, :2])
    br = jnp.minimum(boxes[:, None, 2:], boxes[None, :, 2:])
    wh = jnp.clip(br - tl, 0.0)
    inter = wh[..., 0] * wh[..., 1]
    return inter / (area[:, None] + area[None, :] - inter + 1e-9)


def _nms_keep(boxes, n):
    ious = _iou_matrix(boxes)
    rng = jnp.arange(n)

    def body(i, keep):
        sup = (ious[i] > _NMS_THRESH) & keep[i] & (rng > i)
        return keep & (~sup)

    return jax.lax.fori_loop(0, n, body, jnp.ones((n,), bool))


def _proposal(loc, score, anchor, image_size, scale):
    boxes = _loc2bbox(anchor, loc)
    img_h = image_size[0].astype(jnp.float32)
    img_w = image_size[1].astype(jnp.float32)
    boxes = jnp.stack([
        jnp.clip(boxes[:, 0], 0.0, img_h),
        jnp.clip(boxes[:, 1], 0.0, img_w),
        jnp.clip(boxes[:, 2], 0.0, img_h),
        jnp.clip(boxes[:, 3], 0.0, img_w)], axis=1)
    min_size = _MIN_SIZE * scale
    hs = boxes[:, 2] - boxes[:, 0]
    ws = boxes[:, 3] - boxes[:, 1]
    valid = (hs >= min_size) & (ws >= min_size)
    masked = jnp.where(valid, score, jnp.float32(-1e10))
    _, order = jax.lax.top_k(masked, _PRE_NMS)
    cand = boxes[order]
    keep = _nms_keep(jax.lax.stop_gradient(cand), _PRE_NMS)
    sel = jnp.argsort(jnp.where(keep, 0, 1).astype(jnp.int32))[:_POST_NMS]
    return cand[sel]


def _copy_kernel(x_ref, o_ref):
    o_ref[...] = x_ref[...]


def kernel(x, image_size, anchor, scale, W_conv, b_conv, W_loc, b_loc, W_cls, b_cls):
    batch, _, h, w = x.shape
    hid = jax.nn.relu(_conv2d(x, W_conv, b_conv, 1))
    cls = jax.nn.sigmoid(_conv2d(hid, W_cls, b_cls, 0))
    loc = _conv2d(hid, W_loc, b_loc, 0)
    loc = jnp.transpose(loc, (0, 2, 3, 1)).reshape(batch, -1, 4)
    cls_p = jnp.transpose(cls, (0, 2, 3, 1))
    objectness = cls_p.reshape(batch, h, w, -1, 2)[..., 1].reshape(batch, -1)
    cls_scores = cls_p.reshape(batch, -1, 2)
    rois = []
    roi_indices = []
    for i in range(batch):
        roi = _proposal(loc[i], objectness[i], anchor, image_size, scale)
        roi = pl.pallas_call(
            _copy_kernel,
            out_shape=jax.ShapeDtypeStruct(roi.shape, roi.dtype),
        )(roi)
        rois.append(roi)
        roi_indices.append(jnp.full((roi.shape[0],), i, dtype=jnp.int32))
    return cls_scores, loc, jnp.concatenate(rois, 0), jnp.concatenate(roi_indices, 0)


# speedup vs baseline: 1.0012x; 1.0012x over previous
import jax
import jax.numpy as jnp
import numpy as np
from jax.experimental import pallas as pl

_NMS_THRESH = 0.7
_PRE_NMS = 2000
_POST_NMS = 300
_MIN_SIZE = 16.0

_H, _W = 38, 50
_HP, _WP = 40, 52
_NP = _HP * _WP
_PAD0 = 56
_NB = _PAD0 + _NP + 56


def _trunk_kernel(xb_ref, wr_ref, wh_ref, bc_ref, bh_ref, out_ref):
    acc = jnp.broadcast_to(bc_ref[...], (_NP, 512))
    for t in range(9):
        ky, kx = t // 3, t % 3
        o = (ky - 1) * _WP + (kx - 1)
        acc = acc + jax.lax.dot_general(
            xb_ref[pl.ds(_PAD0 + o, _NP), :], wr_ref[pl.ds(t * 512, 512), :],
            (((1,), (0,)), ((), ())),
            preferred_element_type=jnp.float32,
            precision=jax.lax.Precision.HIGHEST)
    hid = jnp.maximum(acc, 0.0)
    out_ref[...] = jax.lax.dot_general(
        hid, wh_ref[...], (((1,), (0,)), ((), ())),
        preferred_element_type=jnp.float32,
        precision=jax.lax.Precision.HIGHEST) + bh_ref[...]


def _iou_matrix(boxes):
    area = (boxes[:, 2] - boxes[:, 0]) * (boxes[:, 3] - boxes[:, 1])
    tl = jnp.maximum(boxes[:, None, :2], boxes[None, :, :2])
    br = jnp.minimum(boxes[:, None, 2:], boxes[None, :, 2:])
    wh = jnp.clip(br - tl, 0.0)
    inter = wh[..., 0] * wh[..., 1]
    return inter / (area[:, None] + area[None, :] - inter + 1e-9)


def _nms_keep(boxes, n):
    ious = _iou_matrix(boxes)
    rng = jnp.arange(n)

    def body(i, keep):
        sup = (ious[i] > _NMS_THRESH) & keep[i] & (rng > i)
        return keep & (~sup)

    return jax.lax.fori_loop(0, n, body, jnp.ones((n,), bool))


def _loc2bbox(anchor, loc):
    ah = anchor[:, 2] - anchor[:, 0]
    aw = anchor[:, 3] - anchor[:, 1]
    acy = anchor[:, 0] + 0.5 * ah
    acx = anchor[:, 1] + 0.5 * aw
    dy, dx, dh, dw = loc[:, 0], loc[:, 1], loc[:, 2], loc[:, 3]
    cy = dy * ah + acy
    cx = dx * aw + acx
    hh = jnp.exp(dh) * ah
    ww = jnp.exp(dw) * aw
    return jnp.stack([cy - 0.5 * hh, cx - 0.5 * ww, cy + 0.5 * hh, cx + 0.5 * ww], axis=1)


def _proposal(loc, score, anchor, image_size, scale):
    boxes = _loc2bbox(anchor, loc)
    img_h = image_size[0].astype(jnp.float32)
    img_w = image_size[1].astype(jnp.float32)
    boxes = jnp.stack([
        jnp.clip(boxes[:, 0], 0.0, img_h),
        jnp.clip(boxes[:, 1], 0.0, img_w),
        jnp.clip(boxes[:, 2], 0.0, img_h),
        jnp.clip(boxes[:, 3], 0.0, img_w)], axis=1)
    min_size = _MIN_SIZE * scale
    hs = boxes[:, 2] - boxes[:, 0]
    ws = boxes[:, 3] - boxes[:, 1]
    valid = (hs >= min_size) & (ws >= min_size)
    masked = jnp.where(valid, score, jnp.float32(-1e10))
    _, order = jax.lax.top_k(masked, _PRE_NMS)
    cand = boxes[order]
    keep = _nms_keep(jax.lax.stop_gradient(cand), _PRE_NMS)
    sel = jnp.argsort(jnp.where(keep, 0, 1).astype(jnp.int32))[:_POST_NMS]
    return cand[sel]


def kernel(x, image_size, anchor, scale, W_conv, b_conv, W_loc, b_loc, W_cls, b_cls):
    xt = jnp.transpose(x[0], (1, 2, 0))
    xp = jnp.pad(xt, ((1, 1), (1, 1), (0, 0)))
    Xbig = jnp.pad(xp.reshape(_NP, 512), ((_PAD0, _NB - _PAD0 - _NP), (0, 0)))

    Wr = jnp.transpose(W_conv, (2, 3, 1, 0)).reshape(9 * 512, 512)
    Whead = jnp.concatenate([
        jnp.transpose(W_loc[:, :, 0, 0]),
        jnp.transpose(W_cls[:, :, 0, 0]),
        jnp.zeros((512, 128 - 54), jnp.float32)], axis=1)
    bhead = jnp.concatenate([b_loc, b_cls, jnp.zeros((128 - 54,), jnp.float32)])[None, :]

    out = pl.pallas_call(
        _trunk_kernel,
        out_shape=jax.ShapeDtypeStruct((_NP, 128), jnp.float32),
    )(Xbig, Wr, Whead, b_conv[None, :], bhead)

    heads = out.reshape(_HP, _WP, 128)[1:39, 1:51, :54]
    loc = heads[..., :36].reshape(1, -1, 4)
    zcls = heads[..., 36:54]
    cls_p = jax.nn.sigmoid(zcls)[None]
    objectness = cls_p.reshape(1, _H, _W, -1, 2)[..., 1].reshape(1, -1)
    cls_scores = cls_p.reshape(1, -1, 2)

    roi = _proposal(loc[0], objectness[0], anchor, image_size, scale)
    roi_indices = jnp.zeros((roi.shape[0],), dtype=jnp.int32)
    return cls_scores, loc, roi, roi_indices
